# baseline (speedup 1.0000x reference)
import jax
import jax.numpy as jnp
import numpy as np
from jax import lax
from jax.experimental import pallas as pl

EMB = 300
HID = 150


def _lstm_dir(p, prefix, x, lens, reverse):
    Bn, T, _ = x.shape
    Wih = p[prefix + '_wih']
    Whh = p[prefix + '_whh']
    bih = p[prefix + '_bih']
    bhh = p[prefix + '_bhh']
    H = Whh.shape[1]
    t = jnp.arange(T)
    idxc = None
    valid = None
    if reverse:
        idx = lens[:, None] - 1 - t[None, :]
        valid = (idx >= 0).astype(x.dtype)
        idxc = jnp.clip(idx, 0, T - 1)
        x = jnp.take_along_axis(x, idxc[:, :, None], axis=1) * valid[:, :, None]

    def step(carry, xt):
        hh, cc = carry
        g = xt @ Wih.T + hh @ Whh.T + bih + bhh
        i, f, gg, o = jnp.split(g, 4, axis=-1)
        i = jax.nn.sigmoid(i)
        f = jax.nn.sigmoid(f)
        gg = jnp.tanh(gg)
        o = jax.nn.sigmoid(o)
        cc = f * cc + i * gg
        hh = o * jnp.tanh(cc)
        return (hh, cc), hh

    init = (jnp.zeros((Bn, H), x.dtype), jnp.zeros((Bn, H), x.dtype))
    _, hs = lax.scan(step, init, jnp.swapaxes(x, 0, 1))
    hs = jnp.swapaxes(hs, 0, 1)
    if reverse:
        hs = jnp.take_along_axis(hs, idxc[:, :, None], axis=1) * valid[:, :, None]
    mask = (t[None, :] < lens[:, None]).astype(hs.dtype)[:, :, None]
    return hs * mask


def _bilstm(p, x, lens):
    h = x
    for layer in (0, 1):
        fwd = _lstm_dir({'_wih': p['lstm_wih_%df' % layer], '_whh': p['lstm_whh_%df' % layer],
                         '_bih': p['lstm_bih_%df' % layer], '_bhh': p['lstm_bhh_%df' % layer]},
                        '', x=h, lens=lens, reverse=False)
        bwd = _lstm_dir({'_wih': p['lstm_wih_%db' % layer], '_whh': p['lstm_whh_%db' % layer],
                         '_bih': p['lstm_bih_%db' % layer], '_bhh': p['lstm_bhh_%db' % layer]},
                        '', x=h, lens=lens, reverse=True)
        h = jnp.concatenate([fwd, bwd], axis=-1)
    return h


def _gcn_conv(x, edge_index, W, b):
    n = x.shape[0]
    loop = jnp.arange(n, dtype=edge_index.dtype)
    src = jnp.concatenate([edge_index[0], loop])
    dst = jnp.concatenate([edge_index[1], loop])
    w = jnp.ones(src.shape[0], jnp.float32)
    deg = jax.ops.segment_sum(w, dst, num_segments=n)
    dinv = jnp.where(deg > 0, lax.rsqrt(deg), 0.0)
    norm = dinv[src] * dinv[dst]
    h = x @ W.T
    out = jax.ops.segment_sum(h[src] * norm[:, None], dst, num_segments=n)
    return out + b


def kernel(x, sent_x, edge_index, params, pos_table, sent_pos_table):
    p = params
    Bn, Sn, Ln = sent_x.shape
    emb_table = p['embed']
    embed = emb_table[x]
    sent_emb = emb_table[sent_x]
    BS = Bn * Sn
    tokens = sent_x.reshape(BS, Ln)
    emb_flat = sent_emb.reshape(BS, Ln, EMB)
    sentlen = jnp.sum(tokens != 0, axis=-1)
    pos_idx = jnp.arange(1, Ln + 1)[None, :]
    input_pos = jnp.where(pos_idx <= sentlen[:, None], pos_idx, 0)
    pos_emb = pos_table[input_pos]
    conv_in = (emb_flat + pos_emb)[:, None, :, :]
    feats = []
    for h in range(2, 8):
        w = p['conv_w_%d' % h]
        bb = p['conv_b_%d' % h]
        out = lax.conv_general_dilated(conv_in, w, (1, 1), 'VALID', dimension_numbers=('NCHW', 'OIHW', 'NCHW'))
        out = jax.nn.relu(out + bb[None, :, None, None])
        feats.append(jnp.max(out[:, :, :, 0], axis=2))
    ngram = jnp.concatenate(feats, axis=1)
    snode_pos = jnp.tile(jnp.arange(Sn), Bn)
    spos = sent_pos_table[snode_pos]
    cnn_feature = (ngram + spos) @ p['cnn_proj_w'].T + p['cnn_proj_b']
    cnn_feature = cnn_feature.reshape(Bn, Sn, HID)
    glen = jnp.sum(sent_x[:, :, 0] != 0, axis=-1)
    lstm_out = _bilstm(p, cnn_feature, glen)
    lstm_feature = lstm_out @ p['lstm_proj_w'].T + p['lstm_proj_b']
    sent_feat = jnp.concatenate([cnn_feature, lstm_feature], axis=-1)
    xcat = jnp.concatenate([embed, sent_feat], axis=1)
    h0 = (xcat @ p['gc0_w'].T + p['gc0_b']).reshape(-1, HID)
    h1 = jax.nn.elu(_gcn_conv(h0, edge_index, p['gc1_w'], p['gc1_b']))
    out = _gcn_conv(h1, edge_index, p['gc2_w'], p['gc2_b'])
    return out, embed


# speedup vs baseline: 1.6966x; 1.6966x over previous
import functools

import jax
import jax.numpy as jnp
from jax import lax
from jax.experimental import pallas as pl
from jax.experimental.pallas import tpu as pltpu
from jax.experimental.pallas import tpu_sc as plsc

EMB = 300
HID = 150
N_NODES = 10000
N_EDGES = 320000
DPAD = 152
NWORKERS = 32
EPW = N_EDGES // NWORKERS
KCH = 80
NCH = EPW // KCH

_SC_MESH = plsc.VectorSubcoreMesh(core_axis_name="c", subcore_axis_name="s")


@functools.partial(
    pl.kernel,
    out_type=jax.ShapeDtypeStruct((2, N_NODES), jnp.float32),
    mesh=_SC_MESH,
    scratch_types=[
        pltpu.VMEM((NCH, KCH), jnp.int32),
        pltpu.VMEM((KCH,), jnp.float32),
        pltpu.VMEM_SHARED((N_NODES,), jnp.float32),
    ],
)
def _sc_degree(dst_hbm, zero_hbm, out_hbm, dst_v, ones_v, acc):
    c = lax.axis_index("c")
    s = lax.axis_index("s")
    wid = c * 16 + s

    @pl.when(s == 0)
    def _():
        pltpu.sync_copy(zero_hbm, acc)

    pltpu.sync_copy(dst_hbm.at[wid], dst_v)

    def fill(i, carry):
        ones_v[pl.ds(i * 16, 16)] = jnp.ones((16,), jnp.float32)
        return carry

    lax.fori_loop(0, KCH // 16, fill, 0)
    plsc.subcore_barrier()

    def body(j, carry):
        pltpu.sync_copy(ones_v, acc.at[dst_v.at[j]], add=True)
        return carry

    lax.fori_loop(0, NCH, body, 0)
    plsc.subcore_barrier()

    @pl.when(s == 0)
    def _():
        pltpu.sync_copy(acc, out_hbm.at[c])


@functools.partial(
    pl.kernel,
    out_type=jax.ShapeDtypeStruct((2, N_NODES, DPAD), jnp.float32),
    mesh=_SC_MESH,
    compiler_params=pltpu.CompilerParams(use_tc_tiling_on_sc=False),
    scratch_types=[
        pltpu.VMEM((NCH, KCH), jnp.int32),
        pltpu.VMEM((NCH, KCH), jnp.int32),
        pltpu.VMEM((KCH, DPAD), jnp.float32),
        pltpu.VMEM_SHARED((N_NODES, DPAD), jnp.float32),
    ],
)
def _sc_edge_prop(src_hbm, dst_hbm, y_hbm, zero_hbm, out_hbm,
                  src_v, dst_v, rows_v, acc):
    c = lax.axis_index("c")
    s = lax.axis_index("s")
    wid = c * 16 + s

    @pl.when(s == 0)
    def _():
        pltpu.sync_copy(zero_hbm, acc)

    pltpu.sync_copy(src_hbm.at[wid], src_v)
    pltpu.sync_copy(dst_hbm.at[wid], dst_v)
    plsc.subcore_barrier()

    def body(j, carry):
        pltpu.sync_copy(y_hbm.at[src_v.at[j]], rows_v)
        pltpu.sync_copy(rows_v, acc.at[dst_v.at[j]], add=True)
        return carry

    lax.fori_loop(0, NCH, body, 0)
    plsc.subcore_barrier()
    pltpu.sync_copy(acc.at[pl.ds(s * 624, 624)],
                    out_hbm.at[c, pl.ds(s * 624, 624)])

    @pl.when(s == 15)
    def _():
        pltpu.sync_copy(acc.at[pl.ds(9984, N_NODES - 9984)],
                        out_hbm.at[c, pl.ds(9984, N_NODES - 9984)])


def _mm_bias(x, wt, b):
    def kern(x_ref, wt_ref, b_ref, o_ref):
        o_ref[...] = jnp.dot(x_ref[...], wt_ref[...],
                             preferred_element_type=jnp.float32) + b_ref[...]

    return pl.pallas_call(
        kern,
        out_shape=jax.ShapeDtypeStruct((x.shape[0], wt.shape[1]), jnp.float32),
    )(x, wt, b)


def _mm_scale(x, wt, dinv):
    def kern(x_ref, wt_ref, d_ref, o_ref):
        o_ref[...] = jnp.dot(x_ref[...], wt_ref[...],
                             preferred_element_type=jnp.float32) * d_ref[...]

    return pl.pallas_call(
        kern,
        out_shape=jax.ShapeDtypeStruct((x.shape[0], wt.shape[1]), jnp.float32),
    )(x, wt, dinv)


def _dinv_col(deg_t):
    def kern(d_ref, o_ref):
        o_ref[...] = lax.rsqrt(1.0 + jnp.sum(d_ref[...], axis=1, keepdims=True))

    return pl.pallas_call(
        kern,
        out_shape=jax.ShapeDtypeStruct((deg_t.shape[0], 1), jnp.float32),
    )(deg_t)


def _epilogue(p, y, dinv, b, elu):
    def kern(p_ref, y_ref, d_ref, b_ref, o_ref):
        t = (p_ref[0] + p_ref[1] + y_ref[...]) * d_ref[...] + b_ref[...]
        if elu:
            t = jnp.where(t > 0, t, jnp.exp(jnp.minimum(t, 0.0)) - 1.0)
        o_ref[...] = t

    return pl.pallas_call(
        kern,
        out_shape=jax.ShapeDtypeStruct(y.shape, jnp.float32),
    )(p, y, dinv, b)


def _lstm_dir(x, lens, Wih, Whh, bih, bhh, reverse):
    Bn, T, _ = x.shape
    H = Whh.shape[1]
    t = jnp.arange(T)
    idxc = None
    valid = None
    if reverse:
        idx = lens[:, None] - 1 - t[None, :]
        valid = (idx >= 0).astype(x.dtype)
        idxc = jnp.clip(idx, 0, T - 1)
        x = jnp.take_along_axis(x, idxc[:, :, None], axis=1) * valid[:, :, None]

    def step(carry, xt):
        hh, cc = carry
        g = xt @ Wih.T + hh @ Whh.T + bih + bhh
        i, f, gg, o = jnp.split(g, 4, axis=-1)
        i = jax.nn.sigmoid(i)
        f = jax.nn.sigmoid(f)
        gg = jnp.tanh(gg)
        o = jax.nn.sigmoid(o)
        cc = f * cc + i * gg
        hh = o * jnp.tanh(cc)
        return (hh, cc), hh

    init = (jnp.zeros((Bn, H), x.dtype), jnp.zeros((Bn, H), x.dtype))
    _, hs = lax.scan(step, init, jnp.swapaxes(x, 0, 1))
    hs = jnp.swapaxes(hs, 0, 1)
    if reverse:
        hs = jnp.take_along_axis(hs, idxc[:, :, None], axis=1) * valid[:, :, None]
    mask = (t[None, :] < lens[:, None]).astype(hs.dtype)[:, :, None]
    return hs * mask


def _bilstm(p, x, lens):
    h = x
    for layer in (0, 1):
        fwd = _lstm_dir(h, lens, p['lstm_wih_%df' % layer], p['lstm_whh_%df' % layer],
                        p['lstm_bih_%df' % layer], p['lstm_bhh_%df' % layer], False)
        bwd = _lstm_dir(h, lens, p['lstm_wih_%db' % layer], p['lstm_whh_%db' % layer],
                        p['lstm_bih_%db' % layer], p['lstm_bhh_%db' % layer], True)
        h = jnp.concatenate([fwd, bwd], axis=-1)
    return h


def _encoder(p, pos_table, sent_pos_table, x, sent_x):
    Bn, Sn, Ln = sent_x.shape
    emb_table = p['embed']
    embed = emb_table[x]
    sent_emb = emb_table[sent_x]
    BS = Bn * Sn
    tokens = sent_x.reshape(BS, Ln)
    emb_flat = sent_emb.reshape(BS, Ln, EMB)
    sentlen = jnp.sum(tokens != 0, axis=-1)
    pos_idx = jnp.arange(1, Ln + 1)[None, :]
    input_pos = jnp.where(pos_idx <= sentlen[:, None], pos_idx, 0)
    pos_emb = pos_table[input_pos]
    conv_in = (emb_flat + pos_emb)[:, None, :, :]
    feats = []
    for h in range(2, 8):
        w = p['conv_w_%d' % h]
        bb = p['conv_b_%d' % h]
        out = lax.conv_general_dilated(conv_in, w, (1, 1), 'VALID',
                                       dimension_numbers=('NCHW', 'OIHW', 'NCHW'))
        out = jax.nn.relu(out + bb[None, :, None, None])
        feats.append(jnp.max(out[:, :, :, 0], axis=2))
    ngram = jnp.concatenate(feats, axis=1)
    snode_pos = jnp.tile(jnp.arange(Sn), Bn)
    spos = sent_pos_table[snode_pos]
    cnn_feature = (ngram + spos) @ p['cnn_proj_w'].T + p['cnn_proj_b']
    cnn_feature = cnn_feature.reshape(Bn, Sn, HID)
    glen = jnp.sum(sent_x[:, :, 0] != 0, axis=-1)
    lstm_out = _bilstm(p, cnn_feature, glen)
    lstm_feature = lstm_out @ p['lstm_proj_w'].T + p['lstm_proj_b']
    sent_feat = jnp.concatenate([cnn_feature, lstm_feature], axis=-1)
    xcat = jnp.concatenate([embed, sent_feat], axis=1)
    return xcat.reshape(-1, EMB), embed


def kernel(x, sent_x, edge_index, params, pos_table, sent_pos_table):
    p = params

    src_r = edge_index[0].reshape(NWORKERS, NCH, KCH)
    dst_r = edge_index[1].reshape(NWORKERS, NCH, KCH)
    zero_deg = jnp.zeros((N_NODES,), jnp.float32)
    zero_rows = jnp.zeros((N_NODES, DPAD), jnp.float32)

    deg_part = _sc_degree(dst_r, zero_deg)
    dinv = _dinv_col(deg_part.T)

    xcat, embed = _encoder(p, pos_table, sent_pos_table, x, sent_x)

    h0 = _mm_bias(xcat, p['gc0_w'].T, p['gc0_b'][None, :])

    w1t = jnp.zeros((HID, DPAD), jnp.float32).at[:, :HID].set(p['gc1_w'].T)
    w2t = jnp.zeros((DPAD, DPAD), jnp.float32).at[:HID, :HID].set(p['gc2_w'].T)
    b1 = jnp.zeros((1, DPAD), jnp.float32).at[0, :HID].set(p['gc1_b'])
    b2 = jnp.zeros((1, DPAD), jnp.float32).at[0, :HID].set(p['gc2_b'])

    y1 = _mm_scale(h0, w1t, dinv)
    p1 = _sc_edge_prop(src_r, dst_r, y1, zero_rows)
    h1 = _epilogue(p1, y1, dinv, b1, elu=True)

    y2 = _mm_scale(h1, w2t, dinv)
    p2 = _sc_edge_prop(src_r, dst_r, y2, zero_rows)
    out = _epilogue(p2, y2, dinv, b2, elu=False)

    return out[:, :HID], embed
